# baseline (speedup 1.0000x reference)
import functools

import jax
import jax.numpy as jnp
from jax import lax
from jax.experimental import pallas as pl
from jax.experimental.pallas import tpu as pltpu
from jax.experimental.pallas import tpu_sc as plsc

V = 1000
VP = 1024
C = 8
B = 4096

NC = 2
NS = 16
NW = NC * NS
BPW = B // NW
CB = 4
RB = CB * C
NCH = BPW // CB
XN = BPW * C
LANES = 16
SUB = 8


KB = 512
KW = 3 * KB


def _mm_body(embv_ref, w_ref, m_ref):
    kk = pl.program_id(1)
    part = lax.dot_general(
        embv_ref[0], w_ref[...],
        (((1,), (1,)), ((), ())),
        preferred_element_type=jnp.float32)

    @pl.when(kk == 0)
    def _():
        m_ref[0] = part

    @pl.when(kk != 0)
    def _():
        m_ref[0] += part


def _precompute(embv, w_p):
    return pl.pallas_call(
        _mm_body,
        grid=(C, KW // KB),
        in_specs=[
            pl.BlockSpec((1, VP, KB), lambda c, kk: (c, 0, kk)),
            pl.BlockSpec((VP, KB), lambda c, kk: (0, 1 + 2 * c + kk)),
        ],
        out_specs=pl.BlockSpec((1, VP, VP), lambda c, kk: (c, 0, 0)),
        out_shape=jax.ShapeDtypeStruct((C, VP, VP), jnp.float32),
    )(embv, w_p)


def _emb_variants(embed_table):
    variants = []
    for c in range(C):
        lo = KB - 24 * c
        variants.append(jnp.pad(embed_table, ((0, VP - V), (lo, KW - V - lo))))
    return jnp.stack(variants)


_MESH = plsc.VectorSubcoreMesh(core_axis_name="c", subcore_axis_name="s")


PPC = CB * C * SUB


@functools.partial(
    pl.kernel,
    mesh=_MESH,
    out_type=jax.ShapeDtypeStruct((B // SUB, SUB, SUB, 128), jnp.float32),
    scratch_types=[
        pltpu.VMEM((XN // 128, 128), jnp.int32),
        pltpu.VMEM((XN * SUB,), jnp.int32),
        pltpu.VMEM((2, PPC, 128), jnp.float32),
        pltpu.VMEM((2, SUB, SUB, 128), jnp.float32),
        pltpu.VMEM((SUB, 128), jnp.float32),
        pltpu.SemaphoreType.DMA,
        pltpu.SemaphoreType.DMA,
        pltpu.SemaphoreType.DMA,
        pltpu.SemaphoreType.DMA,
    ],
)
def _sc_gather(m_hbm, x_hbm, bias_hbm, out_hbm,
               x_v, idx_v, rows_v, out_v, bias_v,
               gsem0, gsem1, osem0, osem1):
    gsems = (gsem0, gsem1)
    osems = (osem0, osem1)
    wid = lax.axis_index("s") * NC + lax.axis_index("c")
    base_b = wid * BPW

    pltpu.sync_copy(x_hbm.at[pl.ds(wid * (XN // 128), XN // 128)], x_v)
    pltpu.sync_copy(bias_hbm, bias_v)

    iota = lax.iota(jnp.int32, LANES)
    coffs = jnp.bitwise_and(iota, C - 1) * (SUB * SUB * 128)

    @pl.loop(0, XN // 128)
    def _(r):
        for j in range(SUB):
            t = x_v[r, pl.ds(j * LANES, LANES)]
            base = (coffs + jnp.left_shift(jnp.right_shift(t, 3), 6)
                    + jnp.bitwise_and(t, 7))
            dst0 = r * (SUB * 128) + (j // 2) * PPC + (j % 2) * LANES
            for ot in range(SUB):
                idx_v[pl.ds(dst0 + ot * 2 * LANES, LANES)] = base + ot * SUB

    def gather_copy(g, k, half):
        start = pl.multiple_of(g * PPC + half * (PPC // 2), 8)
        return pltpu.make_async_copy(
            m_hbm.at[idx_v.at[pl.ds(start, PPC // 2)]],
            rows_v.at[k, pl.ds(half * (PPC // 2), PPC // 2)], gsems[k])

    base_tt = wid * (BPW // SUB)

    def out_copy(q, o):
        return pltpu.make_async_copy(
            out_v.at[o], out_hbm.at[base_tt + q], osems[o])

    def compute(k, o):
        for e in range(CB):
            @pl.loop(0, SUB)
            def _(ot):
                row = ot * (CB * C) + e * C
                for u in range(128 // LANES):
                    s = pl.ds(u * LANES, LANES)
                    acc = rows_v[k, row, s]
                    for r in range(1, C):
                        acc = acc + rows_v[k, row + r, s]
                    t = acc + bias_v[ot, s]
                    a = jnp.abs(t)
                    ex = jnp.exp(a + a)
                    pos = 1.0 - 2.0 / (ex + 1.0)
                    out_v[o, ot, k * CB + e, s] = jnp.where(t < 0.0, -pos, pos)

    for half in range(2):
        gather_copy(0, 0, half).start()
    for half in range(2):
        gather_copy(1, 1, half).start()

    @pl.loop(0, NCH // 2, step=2)
    def _(q):
        for kq in range(2):
            qq = q + kq

            @pl.when(qq >= 2)
            def _():
                out_copy(qq - 2, kq).wait()

            for k in range(2):
                gg = 2 * qq + k
                for half in range(2):
                    gather_copy(gg, k, half).wait()
                compute(k, kq)

                @pl.when(gg + 2 < NCH)
                def _():
                    for half in range(2):
                        gather_copy(gg + 2, k, half).start()

            out_copy(qq, kq).start()

    for kq in range(2):
        out_copy(NCH // 2 - 2 + kq, kq).wait()


def kernel(x, embed_table, W, b):
    embv = _emb_variants(embed_table)
    w_p = jnp.pad(W, ((0, VP - V), (VP, KB + VP * (C - 1) + KW - VP - C * V)))
    b_p = jnp.pad(b, (0, VP - V)).reshape(SUB, 128)
    m = _precompute(embv, w_p)
    m4 = (m.reshape(C, VP // SUB, SUB, SUB, 128)
          .transpose(0, 1, 3, 2, 4)
          .reshape(C * VP * SUB, 128))
    out4 = _sc_gather(m4, x.reshape(B * C // 128, 128), b_p)
    return out4.transpose(0, 2, 1, 3).reshape(B, VP)[:, :V]
